# baseline (speedup 1.0000x reference)
import functools
import jax
import jax.numpy as jnp
from jax import lax
from jax.experimental import pallas as pl
from jax.experimental.pallas import tpu as pltpu
from jax.experimental.pallas import tpu_sc as plsc

_NC = 2
_NS = 16
_NW = _NC * _NS
_L = 16


def _sc_forward(x_pairs, emb_packed, wcat, B, L, D, V, H, C):
    spw = B // _NW
    groups = spw // _L
    P = D // 2
    LP = L + 1
    ow1, ob1 = 0, H * D
    ow2, ob2 = H * D + H, H * D + H + H * C
    wrows = ob2 + C

    mesh = plsc.VectorSubcoreMesh(
        core_axis_name="c", subcore_axis_name="s",
        num_cores=_NC, num_subcores=_NS,
    )

    @functools.partial(
        pl.kernel,
        out_type=jax.ShapeDtypeStruct((B * C,), jnp.float32),
        mesh=mesh,
        compiler_params=pltpu.CompilerParams(needs_layout_passes=False),
        scratch_types=[
            pltpu.VMEM((spw * LP,), jnp.int32),
            pltpu.VMEM((V * P,), jnp.int32),
            pltpu.VMEM((wrows * _L,), jnp.float32),
            pltpu.VMEM((spw * C,), jnp.float32),
        ],
    )
    def fwd_kernel(x_hbm, emb_hbm, w_hbm, out_hbm, x_v, emb_v, w_v, o_v):
        wid = lax.axis_index("s") * _NC + lax.axis_index("c")
        pltpu.sync_copy(x_hbm.at[pl.ds(wid * spw * LP, spw * LP)], x_v)
        pltpu.sync_copy(emb_hbm, emb_v)
        pltpu.sync_copy(w_hbm, w_v)

        lane = lax.iota(jnp.int32, _L)
        lane_t = lane * LP
        lane_c = lane * C
        vmask = V - 1
        himask = jnp.int32(-65536)

        def group_body(g, carry):
            x_base = g * (_L * LP)

            def tok_body(t, accs):
                v = plsc.load_gather(x_v, [lane_t + (x_base + t)])
                idxp = v & vmask
                for p in range(P):
                    gat = plsc.load_gather(emb_v, [idxp + p * V])
                    lo = plsc.bitcast(gat << 16, jnp.float32)
                    hi = plsc.bitcast(gat & himask, jnp.float32)
                    accs = (accs[:2 * p]
                            + (accs[2 * p] + lo, accs[2 * p + 1] + hi)
                            + accs[2 * p + 2:])
                return accs

            zeros = (jnp.zeros((_L,), jnp.float32),) * D
            h = lax.fori_loop(0, L, tok_body, zeros)

            def mlp_body(j, st):
                outs = st
                z = w_v[pl.ds((ob1 + j) * _L, _L)]
                for d in range(D):
                    z = z + h[d] * w_v[pl.ds((ow1 + j * D + d) * _L, _L)]
                z = jnp.maximum(z, 0.0)
                return tuple(
                    outs[k] + z * w_v[pl.ds((ow2 + j * C + k) * _L, _L)]
                    for k in range(C)
                )

            outs0 = tuple(w_v[pl.ds((ob2 + k) * _L, _L)] for k in range(C))
            outs = lax.fori_loop(0, H, mlp_body, outs0)
            o_base = g * (_L * C)
            for k in range(C):
                plsc.store_scatter(o_v, [lane_c + (o_base + k)], outs[k])
            return carry

        lax.fori_loop(0, groups, group_body, 0)
        pltpu.sync_copy(o_v, out_hbm.at[pl.ds(wid * spw * C, spw * C)])

    return fwd_kernel(x_pairs, emb_packed, wcat)


def kernel(x, embed, W1, b1, W2, b2):
    B, L = x.shape
    V, D = embed.shape
    H = W1.shape[1]
    C = W2.shape[1]

    x_flat = jnp.pad(x.astype(jnp.int32), ((0, 0), (0, 1))).reshape(-1)
    emb_packed = lax.bitcast_convert_type(
        embed.astype(jnp.bfloat16).reshape(V, D // 2, 2),
        jnp.int32).T.reshape(-1)
    wcat = jnp.concatenate(
        [(W1 * (1.0 / L)).T.reshape(-1), b1.reshape(-1),
         W2.reshape(-1), b2.reshape(-1)])
    wsplat = jnp.tile(wcat[:, None], (1, _L)).reshape(-1)

    out = _sc_forward(x_flat, emb_packed, wsplat, B, L, D, V, H, C)
    return out.reshape(B, C)


# speedup vs baseline: 1.0385x; 1.0385x over previous
import functools
import jax
import jax.numpy as jnp
from jax import lax
from jax.experimental import pallas as pl
from jax.experimental.pallas import tpu as pltpu
from jax.experimental.pallas import tpu_sc as plsc

_NC = 2
_NS = 16
_NW = _NC * _NS
_L = 16


def _sc_forward(x_pairs, emb_packed, wcat, B, L, D, V, H, C):
    spw = B // _NW
    groups = spw // _L
    P = D // 2
    ow1, ob1 = 0, H * D
    ow2, ob2 = H * D + H, H * D + H + H * C
    wrows = ob2 + C

    mesh = plsc.VectorSubcoreMesh(
        core_axis_name="c", subcore_axis_name="s",
        num_cores=_NC, num_subcores=_NS,
    )

    @functools.partial(
        pl.kernel,
        out_type=jax.ShapeDtypeStruct((B * C,), jnp.float32),
        mesh=mesh,
        compiler_params=pltpu.CompilerParams(needs_layout_passes=False),
        scratch_types=[
            pltpu.VMEM((spw * L,), jnp.int32),
            pltpu.VMEM((V * P,), jnp.int32),
            pltpu.VMEM((wrows * _L,), jnp.float32),
            pltpu.VMEM((spw * C,), jnp.float32),
        ],
    )
    def fwd_kernel(x_hbm, emb_hbm, w_hbm, out_hbm, x_v, emb_v, w_v, o_v):
        wid = lax.axis_index("s") * _NC + lax.axis_index("c")
        pltpu.sync_copy(x_hbm.at[pl.ds(wid * spw * L, spw * L)], x_v)
        pltpu.sync_copy(emb_hbm, emb_v)
        pltpu.sync_copy(w_hbm, w_v)

        lane = lax.iota(jnp.int32, _L)
        lane_t = lane * L
        lane_c = lane * C
        vmask = V - 1
        himask = jnp.int32(-65536)

        def group_body(g, carry):
            x_base = g * (_L * L)

            def tok_body(t, accs):
                v = plsc.load_gather(x_v, [lane_t + (x_base + t)])
                idxp = v & vmask
                for p in range(P):
                    gat = plsc.load_gather(emb_v, [idxp + p * V])
                    lo = plsc.bitcast(gat << 16, jnp.float32)
                    hi = plsc.bitcast(gat & himask, jnp.float32)
                    accs = (accs[:2 * p]
                            + (accs[2 * p] + lo, accs[2 * p + 1] + hi)
                            + accs[2 * p + 2:])
                return accs

            zeros = (jnp.zeros((_L,), jnp.float32),) * D
            h = lax.fori_loop(0, L, tok_body, zeros)

            def mlp_body(j, st):
                outs = st
                z = w_v[pl.ds((ob1 + j) * _L, _L)]
                for d in range(D):
                    z = z + h[d] * w_v[pl.ds((ow1 + j * D + d) * _L, _L)]
                z = jnp.maximum(z, 0.0)
                return tuple(
                    outs[k] + z * w_v[pl.ds((ow2 + j * C + k) * _L, _L)]
                    for k in range(C)
                )

            outs0 = tuple(w_v[pl.ds((ob2 + k) * _L, _L)] for k in range(C))
            outs = lax.fori_loop(0, H, mlp_body, outs0)
            o_base = g * (_L * C)
            for k in range(C):
                plsc.store_scatter(o_v, [lane_c + (o_base + k)], outs[k])
            return carry

        lax.fori_loop(0, groups, group_body, 0)
        pltpu.sync_copy(o_v, out_hbm.at[pl.ds(wid * spw * C, spw * C)])

    return fwd_kernel(x_pairs, emb_packed, wcat)


def kernel(x, embed, W1, b1, W2, b2):
    B, L = x.shape
    V, D = embed.shape
    H = W1.shape[1]
    C = W2.shape[1]

    x_flat = x.astype(jnp.int32).reshape(-1)
    emb_packed = lax.bitcast_convert_type(
        embed.astype(jnp.bfloat16).reshape(V, D // 2, 2),
        jnp.int32).T.reshape(-1)
    wcat = jnp.concatenate(
        [(W1 * (1.0 / L)).T.reshape(-1), b1.reshape(-1),
         W2.reshape(-1), b2.reshape(-1)])
    wsplat = jnp.tile(wcat[:, None], (1, _L)).reshape(-1)

    out = _sc_forward(x_flat, emb_packed, wsplat, B, L, D, V, H, C)
    return out.reshape(B, C)
